# baseline (speedup 1.0000x reference)
import numpy as np
import jax
import jax.numpy as jnp
from jax.experimental import pallas as pl
from jax.experimental.pallas import tpu as pltpu

B = 64
L = 2048
PL = 32
PS = 16
LW = 8
HG = 128
RH = 128
NC = 24
NPAT = (L - PL) // PS + 1
G = B * NPAT
R = NPAT * PL


def _build_anorm():
    A = np.zeros((PL, PL), np.float32)
    for j in range(PL):
        for i in range(PL):
            if 0 < abs(i - j) <= LW:
                A[j, i] = 1.0
    cnt = np.clip(A.sum(axis=1, keepdims=True), 1.0, None)
    return A / cnt


_ANORM_NP = _build_anorm()


def _bf(x):
    return x.astype(jnp.bfloat16)


def _vn_relu(hc, g, b):
    msq = jnp.sum(hc * hc, axis=-1, keepdims=True) * (1.0 / HG)
    hn = hc * jax.lax.rsqrt(msq + 1e-5) * g + b
    return jnp.maximum(hn, 0.0)


def _gnn_block(x0_ref, anorm_ref, w0_ref, bl0_ref, g0_ref, be0_ref,
               wl1_ref, wr1_ref, bl1_ref, g1_ref, be1_ref,
               wl2_ref, wr2_ref, bl2_ref, g2_ref, be2_ref,
               pf_ref):
    x0 = x0_ref[0]
    A = anorm_ref[...]
    Ab = jnp.broadcast_to(A[None], (NPAT, PL, PL))

    def agg(xb3):
        return jax.lax.dot_general(
            Ab, xb3, (((2,), (1,)), ((0,), (0,))),
            preferred_element_type=jnp.float32)

    x0b = _bf(x0)
    m0 = _bf(agg(x0b))
    u0 = jnp.concatenate([m0, x0b], axis=2).reshape(R, 4)
    h = jnp.dot(u0, w0_ref[...],
                preferred_element_type=jnp.float32) + bl0_ref[...]
    xb = _bf(_vn_relu(h, g0_ref[...], be0_ref[...]))

    xf = None
    for wl_ref, wr_ref, bl_ref, gg_ref, be_ref in (
            (wl1_ref, wr1_ref, bl1_ref, g1_ref, be1_ref),
            (wl2_ref, wr2_ref, bl2_ref, g2_ref, be2_ref)):
        mb = _bf(agg(xb.reshape(NPAT, PL, HG)).reshape(R, HG))
        h = (jnp.dot(mb, wl_ref[...], preferred_element_type=jnp.float32)
             + jnp.dot(xb, wr_ref[...], preferred_element_type=jnp.float32)
             + bl_ref[...])
        xf = xb.astype(jnp.float32) + _vn_relu(h, gg_ref[...], be_ref[...])
        xb = _bf(xf)

    pf_ref[0] = jnp.mean(xf.reshape(NPAT, PL, HG), axis=1)


def _gru_kernel(x_ref, wih0t_ref, whh0t_ref, wih1t_ref, whh1t_ref,
                bih0_ref, bhh0_ref, bih1_ref, bhh1_ref, etf_ref,
                logits_ref, g_ref, pfo_ref, gi_scr, out0_scr):

    def gates(gi, gh, h):
        r = jax.nn.sigmoid(gi[:, :RH] + gh[:, :RH])
        z = jax.nn.sigmoid(gi[:, RH:2 * RH] + gh[:, RH:2 * RH])
        n = jnp.tanh(gi[:, 2 * RH:] + r * gh[:, 2 * RH:])
        return (1.0 - z) * n + z * h

    xin = x_ref[...]
    pfo_ref[...] = xin.reshape(G, HG)
    X = _bf(jnp.transpose(xin, (1, 0, 2)).reshape(NPAT * B, HG))

    gi_scr[...] = jnp.dot(X, wih0t_ref[...],
                          preferred_element_type=jnp.float32) + bih0_ref[...]
    whh0t = whh0t_ref[...]
    bhh0 = bhh0_ref[...]

    def body0(t, h0):
        gi0 = gi_scr[pl.ds(t * B, B), :]
        gh0 = jnp.dot(_bf(h0), whh0t,
                      preferred_element_type=jnp.float32) + bhh0
        h0n = gates(gi0, gh0, h0)
        out0_scr[pl.ds(t * B, B), :] = h0n
        return h0n

    jax.lax.fori_loop(0, NPAT, body0, jnp.zeros((B, RH), jnp.float32))

    gi_scr[...] = jnp.dot(_bf(out0_scr[...]), wih1t_ref[...],
                          preferred_element_type=jnp.float32) + bih1_ref[...]
    whh1t = whh1t_ref[...]
    bhh1 = bhh1_ref[...]

    def body1(t, h1):
        gi1 = gi_scr[pl.ds(t * B, B), :]
        gh1 = jnp.dot(_bf(h1), whh1t,
                      preferred_element_type=jnp.float32) + bhh1
        return gates(gi1, gh1, h1)

    h1 = jax.lax.fori_loop(0, NPAT, body1, jnp.zeros((B, RH), jnp.float32))
    g_ref[...] = h1
    logits_ref[...] = jnp.dot(_bf(h1), etf_ref[...],
                              preferred_element_type=jnp.float32)


def _center(w):
    return w - jnp.mean(w, axis=-1, keepdims=True)


def kernel(iq_signal, params):
    p = params

    def win(s):
        a = s[:, :L - PS].reshape(B, NPAT, PS)
        b2 = s[:, PS:].reshape(B, NPAT, PS)
        return jnp.concatenate([a, b2], axis=2)

    x0 = jnp.stack([win(iq_signal[:, 0, :]), win(iq_signal[:, 1, :])],
                   axis=-1)

    anorm = _bf(jnp.asarray(_ANORM_NP))
    w0 = _bf(jnp.concatenate([_center(p['gnn0_Wl']),
                              _center(p['gnn0_Wr'])], axis=0))

    def full(x):
        nd = x.ndim
        return pl.BlockSpec(x.shape, lambda *a: (0,) * nd)

    def row2(v):
        return v.reshape(1, -1)

    gnn_in = [
        x0, anorm, w0,
        row2(p['gnn0_bl'] - jnp.mean(p['gnn0_bl'])),
        row2(p['ln0_g']), row2(p['ln0_b']),
        _bf(_center(p['gnn1_Wl'])), _bf(_center(p['gnn1_Wr'])),
        row2(p['gnn1_bl'] - jnp.mean(p['gnn1_bl'])),
        row2(p['ln1_g']), row2(p['ln1_b']),
        _bf(_center(p['gnn2_Wl'])), _bf(_center(p['gnn2_Wr'])),
        row2(p['gnn2_bl'] - jnp.mean(p['gnn2_bl'])),
        row2(p['ln2_g']), row2(p['ln2_b']),
    ]
    in_specs = [pl.BlockSpec((1, NPAT, PL, 2), lambda b: (b, 0, 0, 0))]
    in_specs += [full(x) for x in gnn_in[1:]]

    pf = pl.pallas_call(
        _gnn_block,
        grid=(B,),
        in_specs=in_specs,
        out_specs=pl.BlockSpec((1, NPAT, HG), lambda b: (b, 0, 0)),
        out_shape=jax.ShapeDtypeStruct((B, NPAT, HG), jnp.float32),
    )(*gnn_in)

    gru_in = [
        pf,
        _bf(p['gru0_Wih'].T), _bf(p['gru0_Whh'].T),
        _bf(p['gru1_Wih'].T), _bf(p['gru1_Whh'].T),
        row2(p['gru0_bih']), row2(p['gru0_bhh']),
        row2(p['gru1_bih']), row2(p['gru1_bhh']),
        _bf(p['etf']),
    ]
    logits, g, patch_features = pl.pallas_call(
        _gru_kernel,
        in_specs=[full(x) for x in gru_in],
        out_specs=[
            pl.BlockSpec((B, NC), lambda: (0, 0)),
            pl.BlockSpec((B, RH), lambda: (0, 0)),
            pl.BlockSpec((G, HG), lambda: (0, 0)),
        ],
        out_shape=[
            jax.ShapeDtypeStruct((B, NC), jnp.float32),
            jax.ShapeDtypeStruct((B, RH), jnp.float32),
            jax.ShapeDtypeStruct((G, HG), jnp.float32),
        ],
        scratch_shapes=[pltpu.VMEM((NPAT * B, 3 * RH), jnp.float32),
                        pltpu.VMEM((NPAT * B, RH), jnp.float32)],
    )(*gru_in)

    return logits, g, patch_features


# speedup vs baseline: 1.0763x; 1.0763x over previous
import numpy as np
import jax
import jax.numpy as jnp
from jax.experimental import pallas as pl
from jax.experimental.pallas import tpu as pltpu

B = 64
L = 2048
PL = 32
PS = 16
LW = 8
HG = 128
RH = 128
NC = 24
NPAT = (L - PL) // PS + 1
G = B * NPAT
R = NPAT * PL


def _build_anorm():
    A = np.zeros((PL, PL), np.float32)
    for j in range(PL):
        for i in range(PL):
            if 0 < abs(i - j) <= LW:
                A[j, i] = 1.0
    cnt = np.clip(A.sum(axis=1, keepdims=True), 1.0, None)
    return A / cnt


_ANORM_NP = _build_anorm()


def _bf(x):
    return x.astype(jnp.bfloat16)


def _vn_relu(hc, g, b):
    msq = jnp.sum(hc * hc, axis=-1, keepdims=True) * (1.0 / HG)
    hn = hc * jax.lax.rsqrt(msq + 1e-5) * g + b
    return jnp.maximum(hn, 0.0)


def _gnn_block(x0_ref, anorm_ref, w0_ref, bl0_ref, g0_ref, be0_ref,
               wl1_ref, wr1_ref, bl1_ref, g1_ref, be1_ref,
               wl2_ref, wr2_ref, bl2_ref, g2_ref, be2_ref,
               pf_ref):
    x0 = x0_ref[0].reshape(NPAT, PL, 2)
    A = anorm_ref[...]
    Ab = jnp.broadcast_to(A[None], (NPAT, PL, PL))

    def agg(xb3):
        return jax.lax.dot_general(
            Ab, xb3, (((2,), (1,)), ((0,), (0,))),
            preferred_element_type=jnp.float32)

    x0b = _bf(x0)
    m0 = _bf(agg(x0b))
    u0 = jnp.concatenate([m0, x0b], axis=2).reshape(R, 4)
    h = jnp.dot(u0, w0_ref[...],
                preferred_element_type=jnp.float32) + bl0_ref[...]
    xb = _bf(_vn_relu(h, g0_ref[...], be0_ref[...]))

    xf = None
    for wl_ref, wr_ref, bl_ref, gg_ref, be_ref in (
            (wl1_ref, wr1_ref, bl1_ref, g1_ref, be1_ref),
            (wl2_ref, wr2_ref, bl2_ref, g2_ref, be2_ref)):
        mb = _bf(agg(xb.reshape(NPAT, PL, HG)).reshape(R, HG))
        h = (jnp.dot(mb, wl_ref[...], preferred_element_type=jnp.float32)
             + jnp.dot(xb, wr_ref[...], preferred_element_type=jnp.float32)
             + bl_ref[...])
        xf = xb.astype(jnp.float32) + _vn_relu(h, gg_ref[...], be_ref[...])
        xb = _bf(xf)

    pf_ref[0] = jnp.mean(xf.reshape(NPAT, PL, HG), axis=1)


def _gru_kernel(x_ref, wih0t_ref, whh0t_ref, wih1t_ref, whh1t_ref,
                bih0_ref, bhh0_ref, bih1_ref, bhh1_ref, etf_ref,
                logits_ref, g_ref, pfo_ref, gi_scr, out0_scr):

    def gates(gi, gh, h):
        r = jax.nn.sigmoid(gi[:, :RH] + gh[:, :RH])
        z = jax.nn.sigmoid(gi[:, RH:2 * RH] + gh[:, RH:2 * RH])
        n = jnp.tanh(gi[:, 2 * RH:] + r * gh[:, 2 * RH:])
        return (1.0 - z) * n + z * h

    xin = x_ref[...]
    pfo_ref[...] = xin.reshape(G, HG)
    X = _bf(jnp.transpose(xin, (1, 0, 2)).reshape(NPAT * B, HG))

    gi_scr[...] = jnp.dot(X, wih0t_ref[...],
                          preferred_element_type=jnp.float32) + bih0_ref[...]
    whh0t = whh0t_ref[...]
    bhh0 = bhh0_ref[...]

    def body0(t, h0):
        gi0 = gi_scr[pl.ds(t * B, B), :]
        gh0 = jnp.dot(_bf(h0), whh0t,
                      preferred_element_type=jnp.float32) + bhh0
        h0n = gates(gi0, gh0, h0)
        out0_scr[pl.ds(t * B, B), :] = h0n
        return h0n

    jax.lax.fori_loop(0, NPAT, body0, jnp.zeros((B, RH), jnp.float32))

    gi_scr[...] = jnp.dot(_bf(out0_scr[...]), wih1t_ref[...],
                          preferred_element_type=jnp.float32) + bih1_ref[...]
    whh1t = whh1t_ref[...]
    bhh1 = bhh1_ref[...]

    def body1(t, h1):
        gi1 = gi_scr[pl.ds(t * B, B), :]
        gh1 = jnp.dot(_bf(h1), whh1t,
                      preferred_element_type=jnp.float32) + bhh1
        return gates(gi1, gh1, h1)

    h1 = jax.lax.fori_loop(0, NPAT, body1, jnp.zeros((B, RH), jnp.float32))
    g_ref[...] = h1
    logits_ref[...] = jnp.dot(_bf(h1), etf_ref[...],
                              preferred_element_type=jnp.float32)


def _center(w):
    return w - jnp.mean(w, axis=-1, keepdims=True)


def kernel(iq_signal, params):
    p = params

    def win(s):
        a = s[:, :L - PS].reshape(B, NPAT, PS)
        b2 = s[:, PS:].reshape(B, NPAT, PS)
        return jnp.concatenate([a, b2], axis=2)

    x0 = jnp.stack([win(iq_signal[:, 0, :]), win(iq_signal[:, 1, :])],
                   axis=-1).reshape(B, NPAT, PL * 2)

    anorm = _bf(jnp.asarray(_ANORM_NP))
    w0 = _bf(jnp.concatenate([_center(p['gnn0_Wl']),
                              _center(p['gnn0_Wr'])], axis=0))

    def full(x):
        nd = x.ndim
        return pl.BlockSpec(x.shape, lambda *a: (0,) * nd)

    def row2(v):
        return v.reshape(1, -1)

    gnn_in = [
        x0, anorm, w0,
        row2(p['gnn0_bl'] - jnp.mean(p['gnn0_bl'])),
        row2(p['ln0_g']), row2(p['ln0_b']),
        _bf(_center(p['gnn1_Wl'])), _bf(_center(p['gnn1_Wr'])),
        row2(p['gnn1_bl'] - jnp.mean(p['gnn1_bl'])),
        row2(p['ln1_g']), row2(p['ln1_b']),
        _bf(_center(p['gnn2_Wl'])), _bf(_center(p['gnn2_Wr'])),
        row2(p['gnn2_bl'] - jnp.mean(p['gnn2_bl'])),
        row2(p['ln2_g']), row2(p['ln2_b']),
    ]
    in_specs = [pl.BlockSpec((1, NPAT, PL * 2), lambda b: (b, 0, 0))]
    in_specs += [full(x) for x in gnn_in[1:]]

    pf = pl.pallas_call(
        _gnn_block,
        grid=(B,),
        in_specs=in_specs,
        out_specs=pl.BlockSpec((1, NPAT, HG), lambda b: (b, 0, 0)),
        out_shape=jax.ShapeDtypeStruct((B, NPAT, HG), jnp.float32),
    )(*gnn_in)

    gru_in = [
        pf,
        _bf(p['gru0_Wih'].T), _bf(p['gru0_Whh'].T),
        _bf(p['gru1_Wih'].T), _bf(p['gru1_Whh'].T),
        row2(p['gru0_bih']), row2(p['gru0_bhh']),
        row2(p['gru1_bih']), row2(p['gru1_bhh']),
        _bf(p['etf']),
    ]
    logits, g, patch_features = pl.pallas_call(
        _gru_kernel,
        in_specs=[full(x) for x in gru_in],
        out_specs=[
            pl.BlockSpec((B, NC), lambda: (0, 0)),
            pl.BlockSpec((B, RH), lambda: (0, 0)),
            pl.BlockSpec((G, HG), lambda: (0, 0)),
        ],
        out_shape=[
            jax.ShapeDtypeStruct((B, NC), jnp.float32),
            jax.ShapeDtypeStruct((B, RH), jnp.float32),
            jax.ShapeDtypeStruct((G, HG), jnp.float32),
        ],
        scratch_shapes=[pltpu.VMEM((NPAT * B, 3 * RH), jnp.float32),
                        pltpu.VMEM((NPAT * B, RH), jnp.float32)],
    )(*gru_in)

    return logits, g, patch_features
